# baseline (device time: 67062 ns/iter reference)
import math

import jax
import jax.numpy as jnp
from jax import lax
from jax.experimental import pallas as pl
from jax.experimental.pallas import tpu as pltpu

N_DEV = 16
B = 2
SQ = 128
S_TOT = N_DEV * SQ
D = 512
HQ = 4
DH = 64
HD = HQ * DH
KV_W = 2 * B * HD
V0 = B * HD
SCALE = 0.125
LOG_BASE = math.log(10000.0)


def kernel(x, Wq, Wk, Wv, Wo):
    def body(x_ref, wq_ref, wk_ref, wv_ref, wo_ref, out_ref,
             kvbuf, send_sems, recv_sems):
        my_pos = lax.axis_index("i")

        barrier_sem = pltpu.get_barrier_semaphore()
        for d in range(1, N_DEV):
            pl.semaphore_signal(
                barrier_sem, inc=1,
                device_id=((my_pos + d) % N_DEV,),
                device_id_type=pl.DeviceIdType.MESH,
            )
        pl.semaphore_wait(barrier_sem, N_DEV - 1)

        row = lax.broadcasted_iota(jnp.int32, (SQ, HD), 0)
        col = lax.broadcasted_iota(jnp.int32, (SQ, HD), 1)
        d_in_head = col % DH
        expnt = (2 * (d_in_head // 2)).astype(jnp.float32) / DH
        inv = jnp.exp(-expnt * LOG_BASE)
        pos = (my_pos * SQ + row).astype(jnp.float32)
        ang = pos * inv
        cosf = jnp.cos(ang)
        sinf = jnp.sin(ang)

        kk = lax.broadcasted_iota(jnp.int32, (HD, HD), 0)
        jj = lax.broadcasted_iota(jnp.int32, (HD, HD), 1)
        rot = jnp.where(
            (kk == jj + 1) & (jj % 2 == 0), -1.0,
            jnp.where((kk == jj - 1) & (jj % 2 == 1), 1.0, 0.0),
        ).astype(jnp.float32)

        wq = wq_ref[...].astype(jnp.bfloat16)
        wk = wk_ref[...].astype(jnp.bfloat16)
        wv = wv_ref[...].astype(jnp.bfloat16)
        wo = wo_ref[...].astype(jnp.bfloat16)

        xbs = [x_ref[b].astype(jnp.bfloat16) for b in range(B)]
        for b in range(B):
            k_raw = jnp.dot(xbs[b], wk, preferred_element_type=jnp.float32)
            v = jnp.dot(xbs[b], wv, preferred_element_type=jnp.float32)
            k = k_raw * cosf + jnp.dot(
                k_raw, rot, preferred_element_type=jnp.float32) * sinf
            kvbuf[pl.ds(my_pos * SQ, SQ), b * HD:(b + 1) * HD] = k.astype(jnp.bfloat16)
            kvbuf[pl.ds(my_pos * SQ, SQ), V0 + b * HD:V0 + (b + 1) * HD] = (
                v.astype(jnp.bfloat16))

        rdmas = []
        for d in range(1, N_DEV):
            tgt = (my_pos + d) % N_DEV
            rdma = pltpu.make_async_remote_copy(
                src_ref=kvbuf.at[pl.ds(my_pos * SQ, SQ)],
                dst_ref=kvbuf.at[pl.ds(my_pos * SQ, SQ)],
                send_sem=send_sems.at[d - 1],
                recv_sem=recv_sems.at[d - 1],
                device_id=(tgt,),
                device_id_type=pl.DeviceIdType.MESH,
            )
            rdma.start()
            rdmas.append(rdma)

        qs = []
        for b in range(B):
            q_raw = jnp.dot(xbs[b], wq, preferred_element_type=jnp.float32)
            q = q_raw * cosf + jnp.dot(
                q_raw, rot, preferred_element_type=jnp.float32) * sinf
            qs.append(q)

        q_hs = [[qs[b][:, hh * DH:(hh + 1) * DH].astype(jnp.bfloat16)
                 for hh in range(HQ)] for b in range(B)]
        NEG = jnp.float32(-1e30)
        ms = [[jnp.full((SQ, 1), NEG, jnp.float32) for _ in range(HQ)]
              for _ in range(B)]
        ls = [[jnp.zeros((SQ, 1), jnp.float32) for _ in range(HQ)]
              for _ in range(B)]
        accs = [[jnp.zeros((SQ, DH), jnp.float32) for _ in range(HQ)]
                for _ in range(B)]
        for d in range(N_DEV):
            if d > 0:
                rdmas[d - 1].wait_recv()
            slot = (my_pos - d) % N_DEV
            r0 = slot * SQ
            for b in range(B):
                for hh in range(HQ):
                    c0 = b * HD + hh * DH
                    k_h = kvbuf[pl.ds(r0, SQ), c0:c0 + DH]
                    v_h = kvbuf[pl.ds(r0, SQ), V0 + c0:V0 + c0 + DH]
                    s = lax.dot_general(
                        q_hs[b][hh], k_h, (((1,), (1,)), ((), ())),
                        preferred_element_type=jnp.float32) * SCALE
                    m_new = jnp.maximum(ms[b][hh],
                                        jnp.max(s, axis=1, keepdims=True))
                    alpha = jnp.exp(ms[b][hh] - m_new)
                    e = jnp.exp(s - m_new)
                    ls[b][hh] = ls[b][hh] * alpha + jnp.sum(
                        e, axis=1, keepdims=True)
                    accs[b][hh] = accs[b][hh] * alpha + jnp.dot(
                        e.astype(jnp.bfloat16), v_h,
                        preferred_element_type=jnp.float32)
                    ms[b][hh] = m_new

        for b in range(B):
            ctx_b = jnp.concatenate(
                [accs[b][hh] / ls[b][hh] for hh in range(HQ)],
                axis=1).astype(jnp.bfloat16)
            out_ref[b] = jnp.dot(ctx_b, wo, preferred_element_type=jnp.float32)

        for rdma in rdmas:
            rdma.wait_send()

    return pl.pallas_call(
        body,
        out_shape=jax.ShapeDtypeStruct((B, SQ, D), jnp.float32),
        in_specs=[pl.BlockSpec(memory_space=pltpu.VMEM)] * 5,
        out_specs=pl.BlockSpec(memory_space=pltpu.VMEM),
        scratch_shapes=[
            pltpu.VMEM((S_TOT, KV_W), jnp.bfloat16),
            pltpu.SemaphoreType.DMA((N_DEV - 1,)),
            pltpu.SemaphoreType.DMA((N_DEV - 1,)),
        ],
        compiler_params=pltpu.CompilerParams(collective_id=0),
    )(x, Wq, Wk, Wv, Wo)


# device time: 56986 ns/iter; 1.1768x vs baseline; 1.1768x over previous
import math

import jax
import jax.numpy as jnp
from jax import lax
from jax.experimental import pallas as pl
from jax.experimental.pallas import tpu as pltpu

N_DEV = 16
B = 2
SQ = 128
S_TOT = N_DEV * SQ
D = 512
HQ = 4
DH = 64
HD = HQ * DH
KV_W = 2 * B * HD
V0 = B * HD
SCALE = 0.125
LOG_BASE = math.log(10000.0)


def kernel(x, Wq, Wk, Wv, Wo):
    def body(x_ref, wq_ref, wk_ref, wv_ref, wo_ref, out_ref,
             kvbuf, send_sems, recv_sems):
        my_pos = lax.axis_index("i")

        barrier_sem = pltpu.get_barrier_semaphore()
        for d in range(1, N_DEV):
            pl.semaphore_signal(
                barrier_sem, inc=1,
                device_id=((my_pos + d) % N_DEV,),
                device_id_type=pl.DeviceIdType.MESH,
            )
        pl.semaphore_wait(barrier_sem, N_DEV - 1)

        row = lax.broadcasted_iota(jnp.int32, (SQ, HD), 0)
        col = lax.broadcasted_iota(jnp.int32, (SQ, HD), 1)
        d_in_head = col % DH
        expnt = (2 * (d_in_head // 2)).astype(jnp.float32) / DH
        inv = jnp.exp(-expnt * LOG_BASE)
        pos = (my_pos * SQ + row).astype(jnp.float32)
        ang = pos * inv
        cosf = jnp.cos(ang)
        sinf = jnp.sin(ang)

        kk = lax.broadcasted_iota(jnp.int32, (HD, HD), 0)
        jj = lax.broadcasted_iota(jnp.int32, (HD, HD), 1)
        rot = jnp.where(
            (kk == jj + 1) & (jj % 2 == 0), -1.0,
            jnp.where((kk == jj - 1) & (jj % 2 == 1), 1.0, 0.0),
        ).astype(jnp.float32)

        wq = wq_ref[...].astype(jnp.bfloat16)
        wk = wk_ref[...].astype(jnp.bfloat16)
        wv = wv_ref[...].astype(jnp.bfloat16)
        wo = wo_ref[...].astype(jnp.bfloat16)

        xbs = [x_ref[b].astype(jnp.bfloat16) for b in range(B)]
        for b in range(B):
            k_raw = jnp.dot(xbs[b], wk, preferred_element_type=jnp.float32)
            v = jnp.dot(xbs[b], wv, preferred_element_type=jnp.float32)
            k = k_raw * cosf + jnp.dot(
                k_raw, rot, preferred_element_type=jnp.float32) * sinf
            kvbuf[0:SQ, b * HD:(b + 1) * HD] = k.astype(jnp.bfloat16)
            kvbuf[0:SQ, V0 + b * HD:V0 + (b + 1) * HD] = v.astype(jnp.bfloat16)

        rdmas = []
        for d in range(1, N_DEV):
            tgt = (my_pos + d) % N_DEV
            rdma = pltpu.make_async_remote_copy(
                src_ref=kvbuf.at[pl.ds(0, SQ)],
                dst_ref=kvbuf.at[pl.ds(d * SQ, SQ)],
                send_sem=send_sems.at[d - 1],
                recv_sem=recv_sems.at[d - 1],
                device_id=(tgt,),
                device_id_type=pl.DeviceIdType.MESH,
            )
            rdma.start()
            rdmas.append(rdma)

        qs = []
        for b in range(B):
            q_raw = jnp.dot(xbs[b], wq, preferred_element_type=jnp.float32)
            q = q_raw * cosf + jnp.dot(
                q_raw, rot, preferred_element_type=jnp.float32) * sinf
            qs.append(q)

        GROUPS = [(0, 1), (1, 6), (6, 11), (11, 16)]
        q_hs = [[qs[b][:, hh * DH:(hh + 1) * DH].astype(jnp.bfloat16)
                 for hh in range(HQ)] for b in range(B)]
        NEG = jnp.float32(-1e30)
        ms = [[jnp.full((SQ, 1), NEG, jnp.float32) for _ in range(HQ)]
              for _ in range(B)]
        ls = [[jnp.zeros((SQ, 1), jnp.float32) for _ in range(HQ)]
              for _ in range(B)]
        accs = [[jnp.zeros((SQ, DH), jnp.float32) for _ in range(HQ)]
                for _ in range(B)]
        for lo, hi in GROUPS:
            for d in range(max(lo, 1), hi):
                rdmas[d - 1].wait_recv()
            r0, r1 = lo * SQ, hi * SQ
            for b in range(B):
                for hh in range(HQ):
                    c0 = b * HD + hh * DH
                    k_g = kvbuf[r0:r1, c0:c0 + DH]
                    v_g = kvbuf[r0:r1, V0 + c0:V0 + c0 + DH]
                    s = lax.dot_general(
                        q_hs[b][hh], k_g, (((1,), (1,)), ((), ())),
                        preferred_element_type=jnp.float32) * SCALE
                    m_new = jnp.maximum(ms[b][hh],
                                        jnp.max(s, axis=1, keepdims=True))
                    alpha = jnp.exp(ms[b][hh] - m_new)
                    e = jnp.exp(s - m_new)
                    ls[b][hh] = ls[b][hh] * alpha + jnp.sum(
                        e, axis=1, keepdims=True)
                    accs[b][hh] = accs[b][hh] * alpha + jnp.dot(
                        e.astype(jnp.bfloat16), v_g,
                        preferred_element_type=jnp.float32)
                    ms[b][hh] = m_new

        for b in range(B):
            ctx_b = jnp.concatenate(
                [accs[b][hh] / ls[b][hh] for hh in range(HQ)],
                axis=1).astype(jnp.bfloat16)
            out_ref[b] = jnp.dot(ctx_b, wo, preferred_element_type=jnp.float32)

        for rdma in rdmas:
            rdma.wait_send()

    return pl.pallas_call(
        body,
        out_shape=jax.ShapeDtypeStruct((B, SQ, D), jnp.float32),
        in_specs=[pl.BlockSpec(memory_space=pltpu.VMEM)] * 5,
        out_specs=pl.BlockSpec(memory_space=pltpu.VMEM),
        scratch_shapes=[
            pltpu.VMEM((S_TOT, KV_W), jnp.bfloat16),
            pltpu.SemaphoreType.DMA((N_DEV - 1,)),
            pltpu.SemaphoreType.DMA((N_DEV - 1,)),
        ],
        compiler_params=pltpu.CompilerParams(collective_id=0),
    )(x, Wq, Wk, Wv, Wo)


# device time: 42065 ns/iter; 1.5942x vs baseline; 1.3547x over previous
import math

import jax
import jax.numpy as jnp
from jax import lax
from jax.experimental import pallas as pl
from jax.experimental.pallas import tpu as pltpu

N_DEV = 16
Z = 4
P = 4
B = 2
SQ = 128
S_TOT = N_DEV * SQ
D = 512
HQ = 4
DH = 64
HD = HQ * DH
KV_W = 2 * B * HD
V0 = B * HD
SCALE = 0.125
LOG_BASE = math.log(10000.0)

GROUPS = [(0, 1), (1, 4), (4, 7), (7, 10), (10, 13), (13, 16)]


def kernel(x, Wq, Wk, Wv, Wo):
    def body(x_ref, wq_ref, wk_ref, wv_ref, wo_ref, out_ref,
             kvbuf, csend, crecv, psend, precv):
        my_pos = lax.axis_index("i")
        z_me = my_pos // P
        p_me = my_pos % P

        def col_peer(dz):
            return ((z_me + dz) % Z) * P + p_me

        def plane_peer(dp):
            return z_me * P + (p_me + dp) % P

        barrier_sem = pltpu.get_barrier_semaphore()
        for tgt in [col_peer(dz) for dz in (1, 2, 3)] + [
                plane_peer(dp) for dp in (1, 2, 3)]:
            pl.semaphore_signal(
                barrier_sem, inc=1,
                device_id=(tgt,), device_id_type=pl.DeviceIdType.MESH,
            )
        pl.semaphore_wait(barrier_sem, 6)

        row = lax.broadcasted_iota(jnp.int32, (SQ, HD), 0)
        col = lax.broadcasted_iota(jnp.int32, (SQ, HD), 1)
        d_in_head = col % DH
        expnt = (2 * (d_in_head // 2)).astype(jnp.float32) / DH
        inv = jnp.exp(-expnt * LOG_BASE)
        pos = (my_pos * SQ + row).astype(jnp.float32)
        ang = pos * inv
        cosf = jnp.cos(ang)
        sinf = jnp.sin(ang)

        kk = lax.broadcasted_iota(jnp.int32, (HD, HD), 0)
        jj = lax.broadcasted_iota(jnp.int32, (HD, HD), 1)
        rot = jnp.where(
            (kk == jj + 1) & (jj % 2 == 0), -1.0,
            jnp.where((kk == jj - 1) & (jj % 2 == 1), 1.0, 0.0),
        ).astype(jnp.float32)

        wq = wq_ref[...].astype(jnp.bfloat16)
        wk = wk_ref[...].astype(jnp.bfloat16)
        wv = wv_ref[...].astype(jnp.bfloat16)
        wo = wo_ref[...].astype(jnp.bfloat16)

        xbs = [x_ref[b].astype(jnp.bfloat16) for b in range(B)]
        for b in range(B):
            k_raw = jnp.dot(xbs[b], wk, preferred_element_type=jnp.float32)
            v = jnp.dot(xbs[b], wv, preferred_element_type=jnp.float32)
            k = k_raw * cosf + jnp.dot(
                k_raw, rot, preferred_element_type=jnp.float32) * sinf
            kvbuf[0:SQ, b * HD:(b + 1) * HD] = k.astype(jnp.bfloat16)
            kvbuf[0:SQ, V0 + b * HD:V0 + (b + 1) * HD] = v.astype(jnp.bfloat16)

        c_rdmas = {}
        for dz in (1, 2, 3):
            rdma = pltpu.make_async_remote_copy(
                src_ref=kvbuf.at[pl.ds(0, SQ)],
                dst_ref=kvbuf.at[pl.ds((4 - dz) * SQ, SQ)],
                send_sem=csend.at[dz - 1],
                recv_sem=crecv.at[3 - dz],
                device_id=(col_peer(dz),),
                device_id_type=pl.DeviceIdType.MESH,
            )
            rdma.start()
            c_rdmas[dz] = rdma

        p_rdmas = {}

        def forward_to_plane(dz):
            for dp in (1, 2, 3):
                idx = dz * 3 + (3 - dp)
                rdma = pltpu.make_async_remote_copy(
                    src_ref=kvbuf.at[pl.ds(dz * SQ, SQ)],
                    dst_ref=kvbuf.at[pl.ds((4 + idx) * SQ, SQ)],
                    send_sem=psend.at[dz * 3 + dp - 1],
                    recv_sem=precv.at[idx],
                    device_id=(plane_peer(dp),),
                    device_id_type=pl.DeviceIdType.MESH,
                )
                rdma.start()
                p_rdmas[(dz, dp)] = rdma

        forward_to_plane(0)

        qs = []
        for b in range(B):
            q_raw = jnp.dot(xbs[b], wq, preferred_element_type=jnp.float32)
            q = q_raw * cosf + jnp.dot(
                q_raw, rot, preferred_element_type=jnp.float32) * sinf
            qs.append(q)

        for slab in (1, 2, 3):
            c_rdmas[4 - slab].wait_recv()
            forward_to_plane(slab)

        def wait_group(lo, hi):
            for s in range(lo, hi):
                if s == 0:
                    continue
                if s < 4:
                    pass
                else:
                    dz, dp = (s - 4) // 3, 3 - ((s - 4) % 3)
                    p_rdmas[(dz, dp)].wait_recv()

        q_hs = [[qs[b][:, hh * DH:(hh + 1) * DH].astype(jnp.bfloat16)
                 for hh in range(HQ)] for b in range(B)]
        NEG = jnp.float32(-1e30)
        ms = [[jnp.full((SQ, 1), NEG, jnp.float32) for _ in range(HQ)]
              for _ in range(B)]
        ls = [[jnp.zeros((SQ, 1), jnp.float32) for _ in range(HQ)]
              for _ in range(B)]
        accs = [[jnp.zeros((SQ, DH), jnp.float32) for _ in range(HQ)]
                for _ in range(B)]
        for lo, hi in GROUPS:
            wait_group(lo, hi)
            r0, r1 = lo * SQ, hi * SQ
            for b in range(B):
                for hh in range(HQ):
                    c0 = b * HD + hh * DH
                    k_g = kvbuf[r0:r1, c0:c0 + DH]
                    v_g = kvbuf[r0:r1, V0 + c0:V0 + c0 + DH]
                    s = lax.dot_general(
                        q_hs[b][hh], k_g, (((1,), (1,)), ((), ())),
                        preferred_element_type=jnp.float32) * SCALE
                    m_new = jnp.maximum(ms[b][hh],
                                        jnp.max(s, axis=1, keepdims=True))
                    alpha = jnp.exp(ms[b][hh] - m_new)
                    e = jnp.exp(s - m_new)
                    ls[b][hh] = ls[b][hh] * alpha + jnp.sum(
                        e, axis=1, keepdims=True)
                    accs[b][hh] = accs[b][hh] * alpha + jnp.dot(
                        e.astype(jnp.bfloat16), v_g,
                        preferred_element_type=jnp.float32)
                    ms[b][hh] = m_new

        for b in range(B):
            ctx_b = jnp.concatenate(
                [accs[b][hh] / ls[b][hh] for hh in range(HQ)],
                axis=1).astype(jnp.bfloat16)
            out_ref[b] = jnp.dot(ctx_b, wo, preferred_element_type=jnp.float32)

        for rdma in c_rdmas.values():
            rdma.wait_send()
        for rdma in p_rdmas.values():
            rdma.wait_send()

    return pl.pallas_call(
        body,
        out_shape=jax.ShapeDtypeStruct((B, SQ, D), jnp.float32),
        in_specs=[pl.BlockSpec(memory_space=pltpu.VMEM)] * 5,
        out_specs=pl.BlockSpec(memory_space=pltpu.VMEM),
        scratch_shapes=[
            pltpu.VMEM((S_TOT, KV_W), jnp.bfloat16),
            pltpu.SemaphoreType.DMA((3,)),
            pltpu.SemaphoreType.DMA((3,)),
            pltpu.SemaphoreType.DMA((12,)),
            pltpu.SemaphoreType.DMA((12,)),
        ],
        compiler_params=pltpu.CompilerParams(collective_id=0),
    )(x, Wq, Wk, Wv, Wo)


# device time: 36705 ns/iter; 1.8271x vs baseline; 1.1460x over previous
import math

import jax
import jax.numpy as jnp
from jax import lax
from jax.experimental import pallas as pl
from jax.experimental.pallas import tpu as pltpu

N_DEV = 16
Z = 4
P = 4
B = 2
SQ = 128
S_TOT = N_DEV * SQ
D = 512
HQ = 4
DH = 64
HD = HQ * DH
KV_W = 2 * B * HD
V0 = B * HD
SCALE = 0.125
LOG_BASE = math.log(10000.0)

GROUPS = [(0, 1), (1, 4), (4, 7), (7, 10), (10, 13), (13, 16)]


def kernel(x, Wq, Wk, Wv, Wo):
    def body(x_ref, wq_ref, wk_ref, wv_ref, wo_ref, out_ref,
             kbuf, vbuf, csend, crecv, psend, precv):
        my_pos = lax.axis_index("i")
        z_me = my_pos // P
        p_me = my_pos % P

        def col_peer(dz):
            return ((z_me + dz) % Z) * P + p_me

        def plane_peer(dp):
            return z_me * P + (p_me + dp) % P

        barrier_sem = pltpu.get_barrier_semaphore()
        for tgt in [col_peer(dz) for dz in (1, 2, 3)] + [
                plane_peer(dp) for dp in (1, 2, 3)]:
            pl.semaphore_signal(
                barrier_sem, inc=1,
                device_id=(tgt,), device_id_type=pl.DeviceIdType.MESH,
            )
        pl.semaphore_wait(barrier_sem, 6)

        row = lax.broadcasted_iota(jnp.int32, (SQ, HD), 0)
        col = lax.broadcasted_iota(jnp.int32, (SQ, HD), 1)
        d_in_head = col % DH
        expnt = (2 * (d_in_head // 2)).astype(jnp.float32) / DH
        inv = jnp.exp(-expnt * LOG_BASE)
        pos = (my_pos * SQ + row).astype(jnp.float32)
        ang = pos * inv
        cosf = jnp.cos(ang)
        sinf = jnp.sin(ang)

        kk = lax.broadcasted_iota(jnp.int32, (HD, HD), 0)
        jj = lax.broadcasted_iota(jnp.int32, (HD, HD), 1)
        rot = jnp.where(
            (kk == jj + 1) & (jj % 2 == 0), -1.0,
            jnp.where((kk == jj - 1) & (jj % 2 == 1), 1.0, 0.0),
        ).astype(jnp.float32)

        wq = wq_ref[...].astype(jnp.bfloat16)
        wk = wk_ref[...].astype(jnp.bfloat16)
        wv = wv_ref[...].astype(jnp.bfloat16)
        wo = wo_ref[...].astype(jnp.bfloat16)

        xbs = [x_ref[b].astype(jnp.bfloat16) for b in range(B)]
        for b in range(B):
            k_raw = jnp.dot(xbs[b], wk, preferred_element_type=jnp.float32)
            v = jnp.dot(xbs[b], wv, preferred_element_type=jnp.float32)
            k = k_raw * cosf + jnp.dot(
                k_raw, rot, preferred_element_type=jnp.float32) * sinf
            kbuf[0:SQ, b * HD:(b + 1) * HD] = k.astype(jnp.float8_e4m3fn)
            vbuf[0:SQ, b * HD:(b + 1) * HD] = v.astype(jnp.bfloat16)

        c_rdmas = {}
        for dz in (1, 2, 3):
            pair = []
            for t, buf in ((0, kbuf), (1, vbuf)):
                rdma = pltpu.make_async_remote_copy(
                    src_ref=buf.at[pl.ds(0, SQ)],
                    dst_ref=buf.at[pl.ds((4 - dz) * SQ, SQ)],
                    send_sem=csend.at[t, dz - 1],
                    recv_sem=crecv.at[t, 3 - dz],
                    device_id=(col_peer(dz),),
                    device_id_type=pl.DeviceIdType.MESH,
                )
                rdma.start()
                pair.append(rdma)
            c_rdmas[dz] = pair

        p_rdmas = {}

        def forward_to_plane(dz):
            for dp in (1, 2, 3):
                idx = dz * 3 + (3 - dp)
                pair = []
                for t, buf in ((0, kbuf), (1, vbuf)):
                    rdma = pltpu.make_async_remote_copy(
                        src_ref=buf.at[pl.ds(dz * SQ, SQ)],
                        dst_ref=buf.at[pl.ds((4 + idx) * SQ, SQ)],
                        send_sem=psend.at[t, dz * 3 + dp - 1],
                        recv_sem=precv.at[t, idx],
                        device_id=(plane_peer(dp),),
                        device_id_type=pl.DeviceIdType.MESH,
                    )
                    rdma.start()
                    pair.append(rdma)
                p_rdmas[(dz, dp)] = pair

        forward_to_plane(0)

        qs = []
        for b in range(B):
            q_raw = jnp.dot(xbs[b], wq, preferred_element_type=jnp.float32)
            q = q_raw * cosf + jnp.dot(
                q_raw, rot, preferred_element_type=jnp.float32) * sinf
            qs.append(q)

        for slab in (1, 2, 3):
            for rdma in c_rdmas[4 - slab]:
                rdma.wait_recv()
            forward_to_plane(slab)

        def wait_group(lo, hi):
            for s in range(lo, hi):
                if s == 0:
                    continue
                if s < 4:
                    pass
                else:
                    dz, dp = (s - 4) // 3, 3 - ((s - 4) % 3)
                    for rdma in p_rdmas[(dz, dp)]:
                        rdma.wait_recv()

        q_hs = [[qs[b][:, hh * DH:(hh + 1) * DH].astype(jnp.bfloat16)
                 for hh in range(HQ)] for b in range(B)]
        NEG = jnp.float32(-1e30)
        ms = [[jnp.full((SQ, 1), NEG, jnp.float32) for _ in range(HQ)]
              for _ in range(B)]
        ls = [[jnp.zeros((SQ, 1), jnp.float32) for _ in range(HQ)]
              for _ in range(B)]
        accs = [[jnp.zeros((SQ, DH), jnp.float32) for _ in range(HQ)]
                for _ in range(B)]
        for lo, hi in GROUPS:
            wait_group(lo, hi)
            r0, r1 = lo * SQ, hi * SQ
            for b in range(B):
                for hh in range(HQ):
                    c0 = b * HD + hh * DH
                    k_g = kbuf[r0:r1, c0:c0 + DH].astype(jnp.bfloat16)
                    v_g = vbuf[r0:r1, c0:c0 + DH]
                    s = lax.dot_general(
                        q_hs[b][hh], k_g, (((1,), (1,)), ((), ())),
                        preferred_element_type=jnp.float32) * SCALE
                    m_new = jnp.maximum(ms[b][hh],
                                        jnp.max(s, axis=1, keepdims=True))
                    alpha = jnp.exp(ms[b][hh] - m_new)
                    e = jnp.exp(s - m_new)
                    ls[b][hh] = ls[b][hh] * alpha + jnp.sum(
                        e, axis=1, keepdims=True)
                    accs[b][hh] = accs[b][hh] * alpha + jnp.dot(
                        e.astype(jnp.bfloat16), v_g,
                        preferred_element_type=jnp.float32)
                    ms[b][hh] = m_new

        for b in range(B):
            ctx_b = jnp.concatenate(
                [accs[b][hh] / ls[b][hh] for hh in range(HQ)],
                axis=1).astype(jnp.bfloat16)
            out_ref[b] = jnp.dot(ctx_b, wo, preferred_element_type=jnp.float32)

        for pair in c_rdmas.values():
            for rdma in pair:
                rdma.wait_send()
        for pair in p_rdmas.values():
            for rdma in pair:
                rdma.wait_send()

    return pl.pallas_call(
        body,
        out_shape=jax.ShapeDtypeStruct((B, SQ, D), jnp.float32),
        in_specs=[pl.BlockSpec(memory_space=pltpu.VMEM)] * 5,
        out_specs=pl.BlockSpec(memory_space=pltpu.VMEM),
        scratch_shapes=[
            pltpu.VMEM((S_TOT, B * HD), jnp.float8_e4m3fn),
            pltpu.VMEM((S_TOT, B * HD), jnp.bfloat16),
            pltpu.SemaphoreType.DMA((2, 3)),
            pltpu.SemaphoreType.DMA((2, 3)),
            pltpu.SemaphoreType.DMA((2, 12)),
            pltpu.SemaphoreType.DMA((2, 12)),
        ],
        compiler_params=pltpu.CompilerParams(collective_id=0),
    )(x, Wq, Wk, Wv, Wo)
